# Initial kernel scaffold; baseline (speedup 1.0000x reference)
import jax
import jax.numpy as jnp
from jax.experimental import pallas as pl


def kernel(hidden_states, router_W, router_b, expert_W, expert_b):
    raise NotImplementedError("write your pallas kernel here")


# speedup vs baseline: 1.3078x; 1.3078x over previous
import jax
import jax.numpy as jnp
from jax.experimental import pallas as pl
from jax.experimental.pallas import tpu as pltpu

NE = 64
HID = 768
SEQ = 2048
TOPK = 2
TILE = 128
NASSIGN = SEQ * TOPK
PADDED = ((NASSIGN + NE * (TILE - 1) + TILE - 1) // TILE) * TILE
NTILES = PADDED // TILE


def _router_kernel(x_ref, w_ref, b_ref, idx_ref):
    logits = jax.lax.dot_general(
        x_ref[...], w_ref[...], (((1,), (1,)), ((), ())),
        preferred_element_type=jnp.float32) + b_ref[...]
    ids = jax.lax.broadcasted_iota(jnp.int32, logits.shape, 1)
    m1 = jnp.max(logits, axis=-1, keepdims=True)
    i1 = jnp.min(jnp.where(logits == m1, ids, NE), axis=-1)
    masked = jnp.where(ids == i1[:, None], -jnp.inf, logits)
    m2 = jnp.max(masked, axis=-1, keepdims=True)
    i2 = jnp.min(jnp.where(masked == m2, ids, NE), axis=-1)
    idx_ref[...] = jnp.concatenate([i1[:, None], i2[:, None]], axis=1)


def _gmm_kernel(te_ref, x_ref, w_ref, b_ref, y_ref):
    del te_ref
    y_ref[...] = jax.lax.dot_general(
        x_ref[...], w_ref[0], (((1,), (1,)), ((), ())),
        preferred_element_type=jnp.float32) + b_ref[0]


def kernel(hidden_states, router_W, router_b, expert_W, expert_b):
    x = hidden_states.reshape(SEQ, HID)

    idx = pl.pallas_call(
        _router_kernel,
        out_shape=jax.ShapeDtypeStruct((SEQ, TOPK), jnp.int32),
    )(x, router_W, router_b.reshape(1, NE))

    e_flat = idx.reshape(-1)
    order = jnp.argsort(e_flat)
    e_sorted = e_flat[order]
    tok_sorted = (order // TOPK).astype(jnp.int32)
    counts = jnp.bincount(e_flat, length=NE)
    offsets = jnp.concatenate(
        [jnp.zeros((1,), counts.dtype), jnp.cumsum(counts)[:-1]])
    pc = ((counts + TILE - 1) // TILE) * TILE
    cum = jnp.cumsum(pc)
    pad_off = jnp.concatenate([jnp.zeros((1,), cum.dtype), cum[:-1]])
    j = jnp.arange(NASSIGN)
    pos = (pad_off[e_sorted] + (j - offsets[e_sorted])).astype(jnp.int32)
    padded_tok = jnp.full((PADDED,), SEQ, jnp.int32).at[pos].set(tok_sorted)
    tile_expert = jnp.minimum(
        jnp.searchsorted(cum, jnp.arange(NTILES) * TILE, side='right'),
        NE - 1).astype(jnp.int32)

    x_pad = jnp.concatenate([x, jnp.zeros((1, HID), x.dtype)])
    x_sorted = x_pad[padded_tok]

    grid_spec = pltpu.PrefetchScalarGridSpec(
        num_scalar_prefetch=1,
        grid=(NTILES,),
        in_specs=[
            pl.BlockSpec((TILE, HID), lambda i, te: (i, 0)),
            pl.BlockSpec((1, HID, HID), lambda i, te: (te[i], 0, 0)),
            pl.BlockSpec((1, 1, HID), lambda i, te: (te[i], 0, 0)),
        ],
        out_specs=pl.BlockSpec((TILE, HID), lambda i, te: (i, 0)),
    )
    y = pl.pallas_call(
        _gmm_kernel,
        grid_spec=grid_spec,
        out_shape=jax.ShapeDtypeStruct((PADDED, HID), jnp.float32),
    )(tile_expert, x_sorted, expert_W, expert_b.reshape(NE, 1, HID))

    pos_flat = jnp.zeros((NASSIGN,), jnp.int32).at[order].set(pos)
    out = y[pos_flat].reshape(SEQ, TOPK, HID).sum(axis=1)
    return out.reshape(1, SEQ, HID)
